# baseline (speedup 1.0000x reference)
import functools

import jax
import jax.numpy as jnp
from jax import lax
from jax.experimental import pallas as pl
from jax.experimental.pallas import tpu as pltpu
from jax.experimental.pallas import tpu_sc as plsc

VOCAB = 1000000
SEQ_LEN = 200
D_MODEL = 64
BATCH = 4096


def _sc_call(idx_flat, table, pos_enc):
    info = plsc.get_sparse_core_info()
    nc, ns = info.num_cores, info.num_subcores
    nw = nc * ns
    seqs_per_w = BATCH // nw
    rows_per_w = seqs_per_w * SEQ_LEN

    mesh = plsc.VectorSubcoreMesh(core_axis_name="c", subcore_axis_name="s")

    @functools.partial(
        pl.kernel,
        out_type=jax.ShapeDtypeStruct((BATCH * SEQ_LEN, D_MODEL), jnp.float32),
        mesh=mesh,
        scratch_types=[
            pltpu.VMEM((rows_per_w,), jnp.int32),
            pltpu.VMEM((SEQ_LEN, D_MODEL), jnp.float32),
            pltpu.VMEM((SEQ_LEN, D_MODEL), jnp.float32),
            pltpu.SemaphoreType.DMA,
        ],
        compiler_params=pltpu.CompilerParams(use_tc_tiling_on_sc=False),
    )
    def k(idx_hbm, table_hbm, pos_hbm, out_hbm, idx_v, pos_v, row_v, gsem):
        wid = lax.axis_index("s") * nc + lax.axis_index("c")
        base = pl.multiple_of(wid * rows_per_w, rows_per_w)
        pltpu.sync_copy(idx_hbm.at[pl.ds(base, rows_per_w)], idx_v)
        pltpu.sync_copy(pos_hbm, pos_v)

        def seq_body(s, carry):
            off = pl.multiple_of(s * SEQ_LEN, SEQ_LEN)
            g1 = pltpu.async_copy(
                table_hbm.at[idx_v.at[pl.ds(off, 128)]],
                row_v.at[pl.ds(0, 128)], gsem)
            g2 = pltpu.async_copy(
                table_hbm.at[idx_v.at[pl.ds(off + 128, SEQ_LEN - 128)]],
                row_v.at[pl.ds(128, SEQ_LEN - 128)], gsem)
            g1.wait()
            g2.wait()

            def add_body(i, c):
                for j in range(D_MODEL // 16):
                    plsc.addupdate(row_v.at[i, pl.ds(j * 16, 16)],
                                   pos_v[i, pl.ds(j * 16, 16)])
                return c
            lax.fori_loop(0, SEQ_LEN, add_body, 0, unroll=2)

            pltpu.sync_copy(row_v, out_hbm.at[pl.ds(base + off, SEQ_LEN)])
            return carry

        lax.fori_loop(0, seqs_per_w, seq_body, 0)

    return k(idx_flat, table, pos_enc)


def kernel(indices, table, pos_enc):
    idx_flat = indices.reshape(-1).astype(jnp.int32)
    out = _sc_call(idx_flat, table, pos_enc)
    return out.reshape(BATCH, SEQ_LEN, D_MODEL)
